# baseline (speedup 1.0000x reference)
import jax
import jax.numpy as jnp
from jax.experimental import pallas as pl
from jax.experimental.pallas import tpu as pltpu

_R, _C = 5000, 1000
_GRID = 10
_BR = _R // _GRID


def _body(scal_ref, rb_ref, x_ref, e_ref, out_ref, mm_ref, done_ref, rew_ref):
    i = pl.program_id(0)
    before = scal_ref[0]
    after = scal_ref[1]
    x = x_ref[...]
    new = jnp.where(x == before, after, x)
    out_ref[...] = new
    mismatch = jnp.any(new != e_ref[...]).astype(jnp.int32)

    @pl.when(i == 0)
    def _init():
        mm_ref[0] = mismatch

    @pl.when(i > 0)
    def _acc():
        mm_ref[0] = mm_ref[0] | mismatch

    @pl.when(i == _GRID - 1)
    def _final():
        done = mm_ref[0] == 0
        done_ref[0] = done.astype(jnp.int32)
        rew_ref[0] = jnp.where(done, rb_ref[0], jnp.float32(0.0))


def kernel(ids, end_ids, reward_base, before_id, after_id):
    x = ids.reshape(_GRID, _BR, _C)
    e = end_ids.reshape(_GRID, _BR, _C)
    scal = jnp.stack([jnp.asarray(before_id, jnp.int32),
                      jnp.asarray(after_id, jnp.int32)])
    out, _mm, done_i, rew = pl.pallas_call(
        _body,
        grid=(_GRID,),
        in_specs=[
            pl.BlockSpec(memory_space=pltpu.SMEM),
            pl.BlockSpec(memory_space=pltpu.SMEM),
            pl.BlockSpec((1, _BR, _C), lambda i: (i, 0, 0)),
            pl.BlockSpec((1, _BR, _C), lambda i: (i, 0, 0)),
        ],
        out_specs=[
            pl.BlockSpec((1, _BR, _C), lambda i: (i, 0, 0)),
            pl.BlockSpec(memory_space=pltpu.SMEM),
            pl.BlockSpec(memory_space=pltpu.SMEM),
            pl.BlockSpec(memory_space=pltpu.SMEM),
        ],
        out_shape=[
            jax.ShapeDtypeStruct((_GRID, _BR, _C), jnp.int32),
            jax.ShapeDtypeStruct((1,), jnp.int32),
            jax.ShapeDtypeStruct((1,), jnp.int32),
            jax.ShapeDtypeStruct((1,), jnp.float32),
        ],
    )(scal, reward_base, x, e)
    new_ids = out.reshape(ids.shape)
    done = done_i[0].astype(bool)
    reward = rew[0]
    return new_ids, done, reward


# speedup vs baseline: 1.3262x; 1.3262x over previous
import jax
import jax.numpy as jnp
from jax.experimental import pallas as pl
from jax.experimental.pallas import tpu as pltpu

_N, _L = 200000, 25
_GRID = 25
_BR = _N // _GRID


def _body(scal_ref, rb_ref, x_ref, e_ref, out_ref, mm_ref, done_ref, rew_ref):
    i = pl.program_id(0)
    before = scal_ref[0]
    after = scal_ref[1]
    x = x_ref[...]
    new = jnp.where(x == before, after, x)
    out_ref[...] = new
    mismatch = jnp.any(new != e_ref[...]).astype(jnp.int32)

    @pl.when(i == 0)
    def _init():
        mm_ref[0] = mismatch

    @pl.when(i > 0)
    def _acc():
        mm_ref[0] = mm_ref[0] | mismatch

    @pl.when(i == _GRID - 1)
    def _final():
        done = mm_ref[0] == 0
        done_ref[0] = done.astype(jnp.int32)
        rew_ref[0] = jnp.where(done, rb_ref[0], jnp.float32(0.0))


def kernel(ids, end_ids, reward_base, before_id, after_id):
    scal = jnp.stack([jnp.asarray(before_id, jnp.int32),
                      jnp.asarray(after_id, jnp.int32)])
    out, _mm, done_i, rew = pl.pallas_call(
        _body,
        grid=(_GRID,),
        in_specs=[
            pl.BlockSpec(memory_space=pltpu.SMEM),
            pl.BlockSpec(memory_space=pltpu.SMEM),
            pl.BlockSpec((_BR, _L), lambda i: (i, 0)),
            pl.BlockSpec((_BR, _L), lambda i: (i, 0)),
        ],
        out_specs=[
            pl.BlockSpec((_BR, _L), lambda i: (i, 0)),
            pl.BlockSpec(memory_space=pltpu.SMEM),
            pl.BlockSpec(memory_space=pltpu.SMEM),
            pl.BlockSpec(memory_space=pltpu.SMEM),
        ],
        out_shape=[
            jax.ShapeDtypeStruct((_N, _L), jnp.int32),
            jax.ShapeDtypeStruct((1,), jnp.int32),
            jax.ShapeDtypeStruct((1,), jnp.int32),
            jax.ShapeDtypeStruct((1,), jnp.float32),
        ],
    )(scal, reward_base, ids, end_ids)
    done = done_i[0].astype(bool)
    reward = rew[0]
    return out, done, reward


# speedup vs baseline: 9.4385x; 7.1167x over previous
import jax
import jax.numpy as jnp
from jax.experimental import pallas as pl
from jax.experimental.pallas import tpu as pltpu

_N, _L = 200000, 25
_BW = 8192
_GRID = (_N + _BW - 1) // _BW


def _body(scal_ref, rb_ref, x_ref, e_ref, out_ref, mm_ref, done_ref, rew_ref):
    i = pl.program_id(0)
    before = scal_ref[0]
    after = scal_ref[1]
    x = x_ref[...]
    new = jnp.where(x == before, after, x)
    out_ref[...] = new
    col = i * _BW + jax.lax.broadcasted_iota(jnp.int32, (_L, _BW), 1)
    diff = (new != e_ref[...]) & (col < _N)
    mismatch = jnp.any(diff).astype(jnp.int32)

    @pl.when(i == 0)
    def _init():
        mm_ref[0] = mismatch

    @pl.when(i > 0)
    def _acc():
        mm_ref[0] = mm_ref[0] | mismatch

    @pl.when(i == _GRID - 1)
    def _final():
        done = mm_ref[0] == 0
        done_ref[0] = done.astype(jnp.int32)
        rew_ref[0] = jnp.where(done, rb_ref[0], jnp.float32(0.0))


def kernel(ids, end_ids, reward_base, before_id, after_id):
    scal = jnp.stack([jnp.asarray(before_id, jnp.int32),
                      jnp.asarray(after_id, jnp.int32)])
    out, _mm, done_i, rew = pl.pallas_call(
        _body,
        grid=(_GRID,),
        in_specs=[
            pl.BlockSpec(memory_space=pltpu.SMEM),
            pl.BlockSpec(memory_space=pltpu.SMEM),
            pl.BlockSpec((_L, _BW), lambda i: (0, i)),
            pl.BlockSpec((_L, _BW), lambda i: (0, i)),
        ],
        out_specs=[
            pl.BlockSpec((_L, _BW), lambda i: (0, i)),
            pl.BlockSpec(memory_space=pltpu.SMEM),
            pl.BlockSpec(memory_space=pltpu.SMEM),
            pl.BlockSpec(memory_space=pltpu.SMEM),
        ],
        out_shape=[
            jax.ShapeDtypeStruct((_L, _N), jnp.int32),
            jax.ShapeDtypeStruct((1,), jnp.int32),
            jax.ShapeDtypeStruct((1,), jnp.int32),
            jax.ShapeDtypeStruct((1,), jnp.float32),
        ],
    )(scal, reward_base, ids.T, end_ids.T)
    done = done_i[0].astype(bool)
    reward = rew[0]
    return out.T, done, reward


# speedup vs baseline: 10.8802x; 1.1528x over previous
import jax
import jax.numpy as jnp
from jax.experimental import pallas as pl
from jax.experimental.pallas import tpu as pltpu

_N, _L = 200000, 25
_BW = 16384
_GRID = (_N + _BW - 1) // _BW


def _body(scal_ref, rb_ref, x_ref, e_ref, out_ref, mm_ref, done_ref, rew_ref):
    i = pl.program_id(0)
    before = scal_ref[0]
    after = scal_ref[1]
    x = x_ref[...]
    new = jnp.where(x == before, after, x)
    out_ref[...] = new
    d = new != e_ref[...]

    @pl.when(i == 0)
    def _init():
        mm_ref[0] = 0

    @pl.when(i < _GRID - 1)
    def _acc():
        mm_ref[0] = mm_ref[0] | jnp.any(d).astype(jnp.int32)

    @pl.when(i == _GRID - 1)
    def _acc_tail():
        col = i * _BW + jax.lax.broadcasted_iota(jnp.int32, (_L, _BW), 1)
        mm_ref[0] = mm_ref[0] | jnp.any(d & (col < _N)).astype(jnp.int32)

    @pl.when(i == _GRID - 1)
    def _final():
        done = mm_ref[0] == 0
        done_ref[0] = done.astype(jnp.int32)
        rew_ref[0] = jnp.where(done, rb_ref[0], jnp.float32(0.0))


def kernel(ids, end_ids, reward_base, before_id, after_id):
    scal = jnp.stack([jnp.asarray(before_id, jnp.int32),
                      jnp.asarray(after_id, jnp.int32)])
    out, _mm, done_i, rew = pl.pallas_call(
        _body,
        grid=(_GRID,),
        in_specs=[
            pl.BlockSpec(memory_space=pltpu.SMEM),
            pl.BlockSpec(memory_space=pltpu.SMEM),
            pl.BlockSpec((_L, _BW), lambda i: (0, i)),
            pl.BlockSpec((_L, _BW), lambda i: (0, i)),
        ],
        out_specs=[
            pl.BlockSpec((_L, _BW), lambda i: (0, i)),
            pl.BlockSpec(memory_space=pltpu.SMEM),
            pl.BlockSpec(memory_space=pltpu.SMEM),
            pl.BlockSpec(memory_space=pltpu.SMEM),
        ],
        out_shape=[
            jax.ShapeDtypeStruct((_L, _N), jnp.int32),
            jax.ShapeDtypeStruct((1,), jnp.int32),
            jax.ShapeDtypeStruct((1,), jnp.int32),
            jax.ShapeDtypeStruct((1,), jnp.float32),
        ],
    )(scal, reward_base, ids.T, end_ids.T)
    done = done_i[0].astype(bool)
    reward = rew[0]
    return out.T, done, reward


# speedup vs baseline: 11.3995x; 1.0477x over previous
import jax
import jax.numpy as jnp
from jax.experimental import pallas as pl
from jax.experimental.pallas import tpu as pltpu

_N, _L = 200000, 25
_BW = 32768
_GRID = (_N + _BW - 1) // _BW


def _body(scal_ref, rb_ref, x_ref, e_ref, out_ref, mm_ref, done_ref, rew_ref):
    i = pl.program_id(0)
    before = scal_ref[0]
    after = scal_ref[1]
    x = x_ref[...]
    new = jnp.where(x == before, after, x)
    out_ref[...] = new
    d = new != e_ref[...]

    @pl.when(i == 0)
    def _init():
        mm_ref[0] = 0

    @pl.when(i < _GRID - 1)
    def _acc():
        mm_ref[0] = mm_ref[0] | jnp.any(d).astype(jnp.int32)

    @pl.when(i == _GRID - 1)
    def _acc_tail():
        col = i * _BW + jax.lax.broadcasted_iota(jnp.int32, (_L, _BW), 1)
        mm_ref[0] = mm_ref[0] | jnp.any(d & (col < _N)).astype(jnp.int32)

    @pl.when(i == _GRID - 1)
    def _final():
        done = mm_ref[0] == 0
        done_ref[0] = done.astype(jnp.int32)
        rew_ref[0] = jnp.where(done, rb_ref[0], jnp.float32(0.0))


def kernel(ids, end_ids, reward_base, before_id, after_id):
    scal = jnp.stack([jnp.asarray(before_id, jnp.int32),
                      jnp.asarray(after_id, jnp.int32)])
    out, _mm, done_i, rew = pl.pallas_call(
        _body,
        grid=(_GRID,),
        in_specs=[
            pl.BlockSpec(memory_space=pltpu.SMEM),
            pl.BlockSpec(memory_space=pltpu.SMEM),
            pl.BlockSpec((_L, _BW), lambda i: (0, i)),
            pl.BlockSpec((_L, _BW), lambda i: (0, i)),
        ],
        out_specs=[
            pl.BlockSpec((_L, _BW), lambda i: (0, i)),
            pl.BlockSpec(memory_space=pltpu.SMEM),
            pl.BlockSpec(memory_space=pltpu.SMEM),
            pl.BlockSpec(memory_space=pltpu.SMEM),
        ],
        out_shape=[
            jax.ShapeDtypeStruct((_L, _N), jnp.int32),
            jax.ShapeDtypeStruct((1,), jnp.int32),
            jax.ShapeDtypeStruct((1,), jnp.int32),
            jax.ShapeDtypeStruct((1,), jnp.float32),
        ],
    )(scal, reward_base, ids.T, end_ids.T)
    done = done_i[0].astype(bool)
    reward = rew[0]
    return out.T, done, reward


# speedup vs baseline: 11.5997x; 1.0176x over previous
import jax
import jax.numpy as jnp
from jax.experimental import pallas as pl
from jax.experimental.pallas import tpu as pltpu

_N, _L = 200000, 25
_BW = 33408
_GRID = (_N + _BW - 1) // _BW


def _body(scal_ref, rb_ref, x_ref, e_ref, out_ref, mm_ref, done_ref, rew_ref):
    i = pl.program_id(0)
    before = scal_ref[0]
    after = scal_ref[1]
    x = x_ref[...]
    new = jnp.where(x == before, after, x)
    out_ref[...] = new
    d = new != e_ref[...]

    @pl.when(i == 0)
    def _init():
        mm_ref[0] = 0

    @pl.when(i < _GRID - 1)
    def _acc():
        mm_ref[0] = mm_ref[0] | jnp.any(d).astype(jnp.int32)

    @pl.when(i == _GRID - 1)
    def _acc_tail():
        col = i * _BW + jax.lax.broadcasted_iota(jnp.int32, (_L, _BW), 1)
        mm_ref[0] = mm_ref[0] | jnp.any(d & (col < _N)).astype(jnp.int32)

    @pl.when(i == _GRID - 1)
    def _final():
        done = mm_ref[0] == 0
        done_ref[0] = done.astype(jnp.int32)
        rew_ref[0] = jnp.where(done, rb_ref[0], jnp.float32(0.0))


def kernel(ids, end_ids, reward_base, before_id, after_id):
    scal = jnp.stack([jnp.asarray(before_id, jnp.int32),
                      jnp.asarray(after_id, jnp.int32)])
    out, _mm, done_i, rew = pl.pallas_call(
        _body,
        grid=(_GRID,),
        in_specs=[
            pl.BlockSpec(memory_space=pltpu.SMEM),
            pl.BlockSpec(memory_space=pltpu.SMEM),
            pl.BlockSpec((_L, _BW), lambda i: (0, i)),
            pl.BlockSpec((_L, _BW), lambda i: (0, i)),
        ],
        out_specs=[
            pl.BlockSpec((_L, _BW), lambda i: (0, i)),
            pl.BlockSpec(memory_space=pltpu.SMEM),
            pl.BlockSpec(memory_space=pltpu.SMEM),
            pl.BlockSpec(memory_space=pltpu.SMEM),
        ],
        out_shape=[
            jax.ShapeDtypeStruct((_L, _N), jnp.int32),
            jax.ShapeDtypeStruct((1,), jnp.int32),
            jax.ShapeDtypeStruct((1,), jnp.int32),
            jax.ShapeDtypeStruct((1,), jnp.float32),
        ],
    )(scal, reward_base, ids.T, end_ids.T)
    done = done_i[0].astype(bool)
    reward = rew[0]
    return out.T, done, reward
